# Initial kernel scaffold; baseline (speedup 1.0000x reference)
import jax
import jax.numpy as jnp
from jax.experimental import pallas as pl


def kernel(hidden_states, gate_w, w13_weight, w2_weight):
    raise NotImplementedError("write your pallas kernel here")


# speedup vs baseline: 1.5106x; 1.5106x over previous
import functools

import jax
import jax.numpy as jnp
from jax import lax
from jax.experimental import pallas as pl
from jax.experimental.pallas import tpu as pltpu
from jax.experimental.pallas import tpu_sc as plsc

E = 8
K = 2
T = 2048
H = 1024
I = 1792
BLK = 256
NB = 24
NPAD = NB * BLK
IT = 896
NI = I // IT

NW = 32
TPW = T // NW
NCH = TPW // 16


def _routing_body(x_ref, gw_ref, pos_ref, w0b_ref, w1b_ref, be_ref):
    x = x_ref[...]
    gw = gw_ref[...]
    logits = lax.dot_general(x, gw, (((1,), (1,)), ((), ())),
                             preferred_element_type=jnp.float32)
    m = jnp.max(logits, axis=-1, keepdims=True)
    ex = jnp.exp(logits - m)
    p = ex / jnp.sum(ex, axis=-1, keepdims=True)

    eidx = lax.broadcasted_iota(jnp.int32, (T, E), 1)
    m1 = jnp.max(p, axis=-1, keepdims=True)
    idx1 = jnp.min(jnp.where(p == m1, eidx, E), axis=-1, keepdims=True)
    oh0 = eidx == idx1
    pm = jnp.where(oh0, -jnp.inf, p)
    m2 = jnp.max(pm, axis=-1, keepdims=True)
    idx2 = jnp.min(jnp.where(pm == m2, eidx, E), axis=-1, keepdims=True)
    oh1 = eidx == idx2

    s = m1 + m2
    w0 = m1 / s
    w1 = m2 / s

    oh01 = oh0.astype(jnp.int32) + oh1.astype(jnp.int32)
    acc = oh01
    sh = 1
    while sh < T:
        shifted = jnp.concatenate(
            [jnp.zeros((sh, E), jnp.int32), acc[:T - sh]], axis=0)
        acc = acc + shifted
        sh *= 2
    C = acc - oh01

    cnt = jnp.sum(oh01, axis=0, keepdims=True).astype(jnp.float32)
    padded = jnp.ceil(cnt / BLK) * BLK
    tril = (lax.broadcasted_iota(jnp.int32, (E, E), 0)
            <= lax.broadcasted_iota(jnp.int32, (E, E), 1)).astype(jnp.float32)
    cum = lax.dot_general(padded, tril, (((1,), (0,)), ((), ())),
                          preferred_element_type=jnp.float32)
    pad_off = cum - padded

    rank0 = jnp.sum(jnp.where(oh0, C, 0), axis=-1, keepdims=True)
    rank1 = jnp.sum(jnp.where(oh1, C, 0), axis=-1, keepdims=True)
    off0 = jnp.sum(jnp.where(oh0, pad_off, 0.0), axis=-1, keepdims=True)
    off1 = jnp.sum(jnp.where(oh1, pad_off, 0.0), axis=-1, keepdims=True)
    pos0 = rank0 + off0.astype(jnp.int32)
    pos1 = rank1 + off1.astype(jnp.int32)
    pos_ref[...] = jnp.concatenate([pos0, pos1], axis=1)

    w0b_ref[...] = jnp.broadcast_to(w0, (T, 16))
    w1b_ref[...] = jnp.broadcast_to(w1, (T, 16))

    brow = (lax.broadcasted_iota(jnp.int32, (NB, E), 0) * BLK).astype(
        jnp.float32)
    cmp = brow >= jnp.broadcast_to(cum, (NB, E))
    be = jnp.sum(cmp.astype(jnp.int32), axis=-1, keepdims=True)
    be_ref[...] = jnp.minimum(be, E - 1)


def _routing(x, gate_w):
    return pl.pallas_call(
        _routing_body,
        out_shape=[
            jax.ShapeDtypeStruct((T, K), jnp.int32),
            jax.ShapeDtypeStruct((T, 16), jnp.float32),
            jax.ShapeDtypeStruct((T, 16), jnp.float32),
            jax.ShapeDtypeStruct((NB, 1), jnp.int32),
        ],
    )(x, gate_w)


def _disp_body(x_hbm, p0_hbm, p1_hbm, xd_hbm, p0_v, p1_v, rows_v, sem):
    wid = lax.axis_index("s") * 2 + lax.axis_index("c")
    base_t = wid * TPW
    pltpu.sync_copy(p0_hbm.at[wid], p0_v)
    pltpu.sync_copy(p1_hbm.at[wid], p1_v)
    for c in range(NCH):
        pltpu.sync_copy(x_hbm.at[pl.ds(base_t + c * 16, 16)], rows_v)
        pltpu.async_copy(rows_v, xd_hbm.at[p0_v.at[c]], sem).wait()
        pltpu.async_copy(rows_v, xd_hbm.at[p1_v.at[c]], sem).wait()


def _dispatch(x, p0, p1):
    mesh = plsc.VectorSubcoreMesh(core_axis_name="c", subcore_axis_name="s")
    f = functools.partial(
        pl.kernel,
        out_type=jax.ShapeDtypeStruct((NPAD, H), jnp.float32),
        mesh=mesh,
        scratch_types=[
            pltpu.VMEM((NCH, 16), jnp.int32),
            pltpu.VMEM((NCH, 16), jnp.int32),
            pltpu.VMEM((16, H), jnp.float32),
            pltpu.SemaphoreType.DMA,
        ],
    )(_disp_body)
    return f(x, p0, p1)


def _mm_body(be_ref, xd_ref, w13g_ref, w13u_ref, w2_ref, out_ref):
    i = pl.program_id(1)
    xb = xd_ref[...]
    g = lax.dot_general(xb, w13g_ref[0], (((1,), (1,)), ((), ())),
                        preferred_element_type=jnp.float32)
    u = lax.dot_general(xb, w13u_ref[0], (((1,), (1,)), ((), ())),
                        preferred_element_type=jnp.float32)
    act = g * jax.nn.sigmoid(g) * u
    y = lax.dot_general(act, w2_ref[0], (((1,), (1,)), ((), ())),
                        preferred_element_type=jnp.float32)

    @pl.when(i == 0)
    def _():
        out_ref[...] = y

    @pl.when(i != 0)
    def _():
        out_ref[...] = out_ref[...] + y


def _grouped_mm(be, xd, w13, w2):
    grid_spec = pltpu.PrefetchScalarGridSpec(
        num_scalar_prefetch=1,
        grid=(NB, NI),
        in_specs=[
            pl.BlockSpec((BLK, H), lambda b, i, be_r: (b, 0)),
            pl.BlockSpec((1, IT, H), lambda b, i, be_r: (be_r[b], i, 0)),
            pl.BlockSpec((1, IT, H), lambda b, i, be_r: (be_r[b], NI + i, 0)),
            pl.BlockSpec((1, H, IT), lambda b, i, be_r: (be_r[b], 0, i)),
        ],
        out_specs=pl.BlockSpec((BLK, H), lambda b, i, be_r: (b, 0)),
    )
    return pl.pallas_call(
        _mm_body,
        grid_spec=grid_spec,
        out_shape=jax.ShapeDtypeStruct((NPAD, H), jnp.float32),
        compiler_params=pltpu.CompilerParams(
            dimension_semantics=("arbitrary", "arbitrary")),
    )(be, xd, w13, w13, w2)


def _comb_body(yd_hbm, p0_hbm, p1_hbm, w0_hbm, w1_hbm, out_hbm,
               p0_v, p1_v, w0_v, w1_v, r0_v, r1_v, o_v, sem0, sem1):
    wid = lax.axis_index("s") * 2 + lax.axis_index("c")
    base_t = wid * TPW
    pltpu.sync_copy(p0_hbm.at[wid], p0_v)
    pltpu.sync_copy(p1_hbm.at[wid], p1_v)
    pltpu.sync_copy(w0_hbm.at[wid], w0_v)
    pltpu.sync_copy(w1_hbm.at[wid], w1_v)
    for c in range(NCH):
        g0 = pltpu.async_copy(yd_hbm.at[p0_v.at[c]], r0_v, sem0)
        g1 = pltpu.async_copy(yd_hbm.at[p1_v.at[c]], r1_v, sem1)
        g0.wait()
        g1.wait()
        for r in range(16):
            w0 = w0_v[c * 16 + r, :]
            w1 = w1_v[c * 16 + r, :]

            def body(j, _, r=r, w0=w0, w1=w1):
                sl = pl.ds(j * 16, 16)
                o_v[r, sl] = w0 * r0_v[r, sl] + w1 * r1_v[r, sl]
                return 0

            lax.fori_loop(0, H // 16, body, 0)
        pltpu.sync_copy(o_v, out_hbm.at[pl.ds(base_t + c * 16, 16)])


def _combine(yd, p0, p1, w0r, w1r):
    mesh = plsc.VectorSubcoreMesh(core_axis_name="c", subcore_axis_name="s")
    f = functools.partial(
        pl.kernel,
        out_type=jax.ShapeDtypeStruct((T, H), jnp.float32),
        mesh=mesh,
        scratch_types=[
            pltpu.VMEM((NCH, 16), jnp.int32),
            pltpu.VMEM((NCH, 16), jnp.int32),
            pltpu.VMEM((TPW, 16), jnp.float32),
            pltpu.VMEM((TPW, 16), jnp.float32),
            pltpu.VMEM((16, H), jnp.float32),
            pltpu.VMEM((16, H), jnp.float32),
            pltpu.VMEM((16, H), jnp.float32),
            pltpu.SemaphoreType.DMA,
            pltpu.SemaphoreType.DMA,
        ],
    )(_comb_body)
    return f(yd, p0, p1, w0r, w1r)


def kernel(hidden_states, gate_w, w13_weight, w2_weight):
    x = hidden_states
    pos, w0b, w1b, be = _routing(x, gate_w)
    be_flat = be.reshape(NB)
    p0 = pos[:, 0].reshape(NW, NCH, 16)
    p1 = pos[:, 1].reshape(NW, NCH, 16)
    w0r = w0b.reshape(NW, TPW, 16)
    w1r = w1b.reshape(NW, TPW, 16)
    xd = _dispatch(x, p0, p1)
    yd = _grouped_mm(be_flat, xd, w13_weight, w2_weight)
    out = _combine(yd, p0, p1, w0r, w1r)
    return out


# speedup vs baseline: 1.5172x; 1.0044x over previous
import functools

import jax
import jax.numpy as jnp
from jax import lax
from jax.experimental import pallas as pl
from jax.experimental.pallas import tpu as pltpu
from jax.experimental.pallas import tpu_sc as plsc

E = 8
K = 2
T = 2048
H = 1024
I = 1792
BLK = 256
NB = 24
NPAD = NB * BLK
IT = 896
NI = I // IT

NW = 32
TPW = T // NW
NCH = TPW // 16


def _routing_body(x_ref, gw_ref, pos_ref, w0b_ref, w1b_ref, be_ref):
    x = x_ref[...]
    gw = gw_ref[...]
    logits = lax.dot_general(x, gw, (((1,), (1,)), ((), ())),
                             preferred_element_type=jnp.float32)
    m = jnp.max(logits, axis=-1, keepdims=True)
    ex = jnp.exp(logits - m)
    p = ex / jnp.sum(ex, axis=-1, keepdims=True)

    eidx = lax.broadcasted_iota(jnp.int32, (T, E), 1)
    m1 = jnp.max(p, axis=-1, keepdims=True)
    idx1 = jnp.min(jnp.where(p == m1, eidx, E), axis=-1, keepdims=True)
    oh0 = eidx == idx1
    pm = jnp.where(oh0, -jnp.inf, p)
    m2 = jnp.max(pm, axis=-1, keepdims=True)
    idx2 = jnp.min(jnp.where(pm == m2, eidx, E), axis=-1, keepdims=True)
    oh1 = eidx == idx2

    s = m1 + m2
    w0 = m1 / s
    w1 = m2 / s

    oh01 = oh0.astype(jnp.int32) + oh1.astype(jnp.int32)
    acc = oh01
    sh = 1
    while sh < T:
        shifted = jnp.concatenate(
            [jnp.zeros((sh, E), jnp.int32), acc[:T - sh]], axis=0)
        acc = acc + shifted
        sh *= 2
    C = acc - oh01

    cnt = jnp.sum(oh01, axis=0, keepdims=True).astype(jnp.float32)
    padded = jnp.ceil(cnt / BLK) * BLK
    tril = (lax.broadcasted_iota(jnp.int32, (E, E), 0)
            <= lax.broadcasted_iota(jnp.int32, (E, E), 1)).astype(jnp.float32)
    cum = lax.dot_general(padded, tril, (((1,), (0,)), ((), ())),
                          preferred_element_type=jnp.float32)
    pad_off = cum - padded

    rank0 = jnp.sum(jnp.where(oh0, C, 0), axis=-1, keepdims=True)
    rank1 = jnp.sum(jnp.where(oh1, C, 0), axis=-1, keepdims=True)
    off0 = jnp.sum(jnp.where(oh0, pad_off, 0.0), axis=-1, keepdims=True)
    off1 = jnp.sum(jnp.where(oh1, pad_off, 0.0), axis=-1, keepdims=True)
    pos0 = rank0 + off0.astype(jnp.int32)
    pos1 = rank1 + off1.astype(jnp.int32)
    pos_ref[...] = jnp.concatenate([pos0, pos1], axis=1)

    w0b_ref[...] = jnp.broadcast_to(w0, (T, 16))
    w1b_ref[...] = jnp.broadcast_to(w1, (T, 16))

    brow = (lax.broadcasted_iota(jnp.int32, (NB + 8, E), 0) * BLK).astype(
        jnp.float32)
    cmp = brow >= jnp.broadcast_to(cum, (NB + 8, E))
    be = jnp.sum(cmp.astype(jnp.int32), axis=-1, keepdims=True)
    eiota = lax.broadcasted_iota(jnp.int32, (1, E), 1)
    maxe = jnp.max(jnp.where(cnt > 0, eiota, 0), axis=-1, keepdims=True)
    nact = (cum[:, E - 1:E] / BLK).astype(jnp.int32)
    biota = lax.broadcasted_iota(jnp.int32, (NB + 8, 1), 0)
    be = jnp.minimum(be, jnp.broadcast_to(maxe, (NB + 8, 1)))
    be_ref[...] = jnp.where(biota == NB, jnp.broadcast_to(nact, (NB + 8, 1)),
                            be)


def _routing(x, gate_w):
    return pl.pallas_call(
        _routing_body,
        out_shape=[
            jax.ShapeDtypeStruct((T, K), jnp.int32),
            jax.ShapeDtypeStruct((T, 16), jnp.float32),
            jax.ShapeDtypeStruct((T, 16), jnp.float32),
            jax.ShapeDtypeStruct((NB + 8, 1), jnp.int32),
        ],
    )(x, gate_w)


def _disp_body(x_hbm, p0_hbm, p1_hbm, xd_hbm, p0_v, p1_v, rows_v, sem):
    wid = lax.axis_index("s") * 2 + lax.axis_index("c")
    base_t = wid * TPW
    pltpu.sync_copy(p0_hbm.at[wid], p0_v)
    pltpu.sync_copy(p1_hbm.at[wid], p1_v)
    for c in range(NCH):
        pltpu.sync_copy(x_hbm.at[pl.ds(base_t + c * 16, 16)], rows_v)
        pltpu.async_copy(rows_v, xd_hbm.at[p0_v.at[c]], sem).wait()
        pltpu.async_copy(rows_v, xd_hbm.at[p1_v.at[c]], sem).wait()


def _dispatch(x, p0, p1):
    mesh = plsc.VectorSubcoreMesh(core_axis_name="c", subcore_axis_name="s")
    f = functools.partial(
        pl.kernel,
        out_type=jax.ShapeDtypeStruct((NPAD, H), jnp.float32),
        mesh=mesh,
        scratch_types=[
            pltpu.VMEM((NCH, 16), jnp.int32),
            pltpu.VMEM((NCH, 16), jnp.int32),
            pltpu.VMEM((16, H), jnp.float32),
            pltpu.SemaphoreType.DMA,
        ],
    )(_disp_body)
    return f(x, p0, p1)


def _mm_body(be_ref, xd_ref, w13g_ref, w13u_ref, w2_ref, out_ref):
    b = pl.program_id(0)
    i = pl.program_id(1)
    nact = be_ref[NB]

    @pl.when(b < nact)
    def _():
        xb = xd_ref[...].astype(jnp.bfloat16)
        g = lax.dot_general(
            xb, w13g_ref[0].astype(jnp.bfloat16), (((1,), (1,)), ((), ())),
            preferred_element_type=jnp.float32)
        u = lax.dot_general(
            xb, w13u_ref[0].astype(jnp.bfloat16), (((1,), (1,)), ((), ())),
            preferred_element_type=jnp.float32)
        act = (g * jax.nn.sigmoid(g) * u).astype(jnp.bfloat16)
        y = lax.dot_general(
            act, w2_ref[0].astype(jnp.bfloat16), (((1,), (1,)), ((), ())),
            preferred_element_type=jnp.float32)

        @pl.when(i == 0)
        def _():
            out_ref[...] = y

        @pl.when(i != 0)
        def _():
            out_ref[...] = out_ref[...] + y


def _grouped_mm(be, xd, w13, w2):
    grid_spec = pltpu.PrefetchScalarGridSpec(
        num_scalar_prefetch=1,
        grid=(NB, NI),
        in_specs=[
            pl.BlockSpec((BLK, H), lambda b, i, be_r: (b, 0)),
            pl.BlockSpec((1, IT, H), lambda b, i, be_r: (be_r[b], i, 0)),
            pl.BlockSpec((1, IT, H), lambda b, i, be_r: (be_r[b], NI + i, 0)),
            pl.BlockSpec((1, H, IT), lambda b, i, be_r: (be_r[b], 0, i)),
        ],
        out_specs=pl.BlockSpec((BLK, H), lambda b, i, be_r: (b, 0)),
    )
    return pl.pallas_call(
        _mm_body,
        grid_spec=grid_spec,
        out_shape=jax.ShapeDtypeStruct((NPAD, H), jnp.float32),
        compiler_params=pltpu.CompilerParams(
            dimension_semantics=("arbitrary", "arbitrary")),
    )(be, xd, w13, w13, w2)


def _comb_body(yd_hbm, p0_hbm, p1_hbm, w0_hbm, w1_hbm, out_hbm,
               p0_v, p1_v, w0_v, w1_v, r0_v, r1_v, o_v, sem0, sem1):
    wid = lax.axis_index("s") * 2 + lax.axis_index("c")
    base_t = wid * TPW
    pltpu.sync_copy(p0_hbm.at[wid], p0_v)
    pltpu.sync_copy(p1_hbm.at[wid], p1_v)
    pltpu.sync_copy(w0_hbm.at[wid], w0_v)
    pltpu.sync_copy(w1_hbm.at[wid], w1_v)
    for c in range(NCH):
        g0 = pltpu.async_copy(yd_hbm.at[p0_v.at[c]], r0_v, sem0)
        g1 = pltpu.async_copy(yd_hbm.at[p1_v.at[c]], r1_v, sem1)
        g0.wait()
        g1.wait()
        for r in range(16):
            w0 = w0_v[c * 16 + r, :]
            w1 = w1_v[c * 16 + r, :]

            def body(j, _, r=r, w0=w0, w1=w1):
                sl = pl.ds(j * 16, 16)
                o_v[r, sl] = w0 * r0_v[r, sl] + w1 * r1_v[r, sl]
                return 0

            lax.fori_loop(0, H // 16, body, 0)
        pltpu.sync_copy(o_v, out_hbm.at[pl.ds(base_t + c * 16, 16)])


def _combine(yd, p0, p1, w0r, w1r):
    mesh = plsc.VectorSubcoreMesh(core_axis_name="c", subcore_axis_name="s")
    f = functools.partial(
        pl.kernel,
        out_type=jax.ShapeDtypeStruct((T, H), jnp.float32),
        mesh=mesh,
        scratch_types=[
            pltpu.VMEM((NCH, 16), jnp.int32),
            pltpu.VMEM((NCH, 16), jnp.int32),
            pltpu.VMEM((TPW, 16), jnp.float32),
            pltpu.VMEM((TPW, 16), jnp.float32),
            pltpu.VMEM((16, H), jnp.float32),
            pltpu.VMEM((16, H), jnp.float32),
            pltpu.VMEM((16, H), jnp.float32),
            pltpu.SemaphoreType.DMA,
            pltpu.SemaphoreType.DMA,
        ],
    )(_comb_body)
    return f(yd, p0, p1, w0r, w1r)


def kernel(hidden_states, gate_w, w13_weight, w2_weight):
    x = hidden_states
    pos, w0b, w1b, be = _routing(x, gate_w)
    be_flat = be.reshape(NB + 8)
    p0 = pos[:, 0].reshape(NW, NCH, 16)
    p1 = pos[:, 1].reshape(NW, NCH, 16)
    w0r = w0b.reshape(NW, TPW, 16)
    w1r = w1b.reshape(NW, TPW, 16)
    xd = _dispatch(x, p0, p1)
    yd = _grouped_mm(be_flat, xd, w13_weight, w2_weight)
    out = _combine(yd, p0, p1, w0r, w1r)
    return out


# speedup vs baseline: 2.0077x; 1.3232x over previous
import functools

import jax
import jax.numpy as jnp
from jax import lax
from jax.experimental import pallas as pl
from jax.experimental.pallas import tpu as pltpu
from jax.experimental.pallas import tpu_sc as plsc

E = 8
K = 2
T = 2048
H = 1024
I = 1792
BLK = 256
NB = 24
NPAD = NB * BLK
IT = 896
NI = I // IT

NW = 32
TPW = T // NW
NCH = TPW // 16


def _routing_body(x_ref, gw_ref, pos_ref, w0b_ref, w1b_ref, be_ref):
    x = x_ref[...]
    gw = gw_ref[...]
    logits = lax.dot_general(x, gw, (((1,), (1,)), ((), ())),
                             preferred_element_type=jnp.float32)
    m = jnp.max(logits, axis=-1, keepdims=True)
    ex = jnp.exp(logits - m)
    p = ex / jnp.sum(ex, axis=-1, keepdims=True)

    eidx = lax.broadcasted_iota(jnp.int32, (T, E), 1)
    m1 = jnp.max(p, axis=-1, keepdims=True)
    idx1 = jnp.min(jnp.where(p == m1, eidx, E), axis=-1, keepdims=True)
    oh0 = eidx == idx1
    pm = jnp.where(oh0, -jnp.inf, p)
    m2 = jnp.max(pm, axis=-1, keepdims=True)
    idx2 = jnp.min(jnp.where(pm == m2, eidx, E), axis=-1, keepdims=True)
    oh1 = eidx == idx2

    s = m1 + m2
    w0 = m1 / s
    w1 = m2 / s

    oh01 = oh0.astype(jnp.int32) + oh1.astype(jnp.int32)
    acc = oh01
    sh = 1
    while sh < T:
        shifted = jnp.concatenate(
            [jnp.zeros((sh, E), jnp.int32), acc[:T - sh]], axis=0)
        acc = acc + shifted
        sh *= 2
    C = acc - oh01

    cnt = jnp.sum(oh01, axis=0, keepdims=True).astype(jnp.float32)
    padded = jnp.ceil(cnt / BLK) * BLK
    tril = (lax.broadcasted_iota(jnp.int32, (E, E), 0)
            <= lax.broadcasted_iota(jnp.int32, (E, E), 1)).astype(jnp.float32)
    cum = lax.dot_general(padded, tril, (((1,), (0,)), ((), ())),
                          preferred_element_type=jnp.float32)
    pad_off = cum - padded

    rank0 = jnp.sum(jnp.where(oh0, C, 0), axis=-1, keepdims=True)
    rank1 = jnp.sum(jnp.where(oh1, C, 0), axis=-1, keepdims=True)
    off0 = jnp.sum(jnp.where(oh0, pad_off, 0.0), axis=-1, keepdims=True)
    off1 = jnp.sum(jnp.where(oh1, pad_off, 0.0), axis=-1, keepdims=True)
    pos0 = rank0 + off0.astype(jnp.int32)
    pos1 = rank1 + off1.astype(jnp.int32)
    pos_ref[...] = jnp.concatenate([pos0, pos1], axis=1)

    w0b_ref[...] = jnp.broadcast_to(w0, (T, 16))
    w1b_ref[...] = jnp.broadcast_to(w1, (T, 16))

    brow = (lax.broadcasted_iota(jnp.int32, (NB + 8, E), 0) * BLK).astype(
        jnp.float32)
    cmp = brow >= jnp.broadcast_to(cum, (NB + 8, E))
    be = jnp.sum(cmp.astype(jnp.int32), axis=-1, keepdims=True)
    eiota = lax.broadcasted_iota(jnp.int32, (1, E), 1)
    maxe = jnp.max(jnp.where(cnt > 0, eiota, 0), axis=-1, keepdims=True)
    nact = (cum[:, E - 1:E] / BLK).astype(jnp.int32)
    biota = lax.broadcasted_iota(jnp.int32, (NB + 8, 1), 0)
    be = jnp.minimum(be, jnp.broadcast_to(maxe, (NB + 8, 1)))
    be_ref[...] = jnp.where(biota == NB, jnp.broadcast_to(nact, (NB + 8, 1)),
                            be)


def _routing(x, gate_w):
    return pl.pallas_call(
        _routing_body,
        out_shape=[
            jax.ShapeDtypeStruct((T, K), jnp.int32),
            jax.ShapeDtypeStruct((T, 16), jnp.float32),
            jax.ShapeDtypeStruct((T, 16), jnp.float32),
            jax.ShapeDtypeStruct((NB + 8, 1), jnp.int32),
        ],
    )(x, gate_w)


def _disp_body(x_hbm, p0_hbm, p1_hbm, xd_hbm, p0_v, p1_v, rows_v, sem):
    wid = lax.axis_index("s") * 2 + lax.axis_index("c")
    base_t = wid * TPW
    pltpu.sync_copy(p0_hbm.at[wid], p0_v)
    pltpu.sync_copy(p1_hbm.at[wid], p1_v)
    for c in range(NCH):
        pltpu.sync_copy(x_hbm.at[pl.ds(base_t + c * 16, 16)], rows_v)
        pltpu.async_copy(rows_v, xd_hbm.at[p0_v.at[c]], sem).wait()
        pltpu.async_copy(rows_v, xd_hbm.at[p1_v.at[c]], sem).wait()


def _dispatch(x, p0, p1):
    mesh = plsc.VectorSubcoreMesh(core_axis_name="c", subcore_axis_name="s")
    f = functools.partial(
        pl.kernel,
        out_type=jax.ShapeDtypeStruct((NPAD, H), jnp.float32),
        mesh=mesh,
        scratch_types=[
            pltpu.VMEM((NCH, 16), jnp.int32),
            pltpu.VMEM((NCH, 16), jnp.int32),
            pltpu.VMEM((16, H), jnp.float32),
            pltpu.SemaphoreType.DMA,
        ],
    )(_disp_body)
    return f(x, p0, p1)


def _mm_body(be_ref, xd_ref, w13g_ref, w13u_ref, w2_ref, out_ref):
    b = pl.program_id(0)
    nact = be_ref[NB]

    @pl.when(b < nact)
    def _():
        xb = xd_ref[...].astype(jnp.bfloat16)
        g = lax.dot_general(
            xb, w13g_ref[0].astype(jnp.bfloat16), (((1,), (1,)), ((), ())),
            preferred_element_type=jnp.float32)
        u = lax.dot_general(
            xb, w13u_ref[0].astype(jnp.bfloat16), (((1,), (1,)), ((), ())),
            preferred_element_type=jnp.float32)
        act = (g * jax.nn.sigmoid(g) * u).astype(jnp.bfloat16)
        out_ref[...] = lax.dot_general(
            act, w2_ref[0].astype(jnp.bfloat16), (((1,), (1,)), ((), ())),
            preferred_element_type=jnp.float32)


def _grouped_mm(be, xd, w13, w2):
    grid_spec = pltpu.PrefetchScalarGridSpec(
        num_scalar_prefetch=1,
        grid=(NB,),
        in_specs=[
            pl.BlockSpec((BLK, H), lambda b, be_r: (b, 0)),
            pl.BlockSpec((1, I, H), lambda b, be_r: (be_r[b], 0, 0)),
            pl.BlockSpec((1, I, H), lambda b, be_r: (be_r[b], 1, 0)),
            pl.BlockSpec((1, H, I), lambda b, be_r: (be_r[b], 0, 0)),
        ],
        out_specs=pl.BlockSpec((BLK, H), lambda b, be_r: (b, 0)),
    )
    return pl.pallas_call(
        _mm_body,
        grid_spec=grid_spec,
        out_shape=jax.ShapeDtypeStruct((NPAD, H), jnp.float32),
        compiler_params=pltpu.CompilerParams(
            dimension_semantics=("arbitrary",)),
    )(be, xd, w13, w13, w2)


def _comb_body(yd_hbm, p0_hbm, p1_hbm, w0_hbm, w1_hbm, out_hbm,
               p0_v, p1_v, w0_v, w1_v, r0_v, r1_v, o_v, sem0, sem1):
    wid = lax.axis_index("s") * 2 + lax.axis_index("c")
    base_t = wid * TPW
    pltpu.sync_copy(p0_hbm.at[wid], p0_v)
    pltpu.sync_copy(p1_hbm.at[wid], p1_v)
    pltpu.sync_copy(w0_hbm.at[wid], w0_v)
    pltpu.sync_copy(w1_hbm.at[wid], w1_v)
    for c in range(NCH):
        g0 = pltpu.async_copy(yd_hbm.at[p0_v.at[c]], r0_v, sem0)
        g1 = pltpu.async_copy(yd_hbm.at[p1_v.at[c]], r1_v, sem1)
        g0.wait()
        g1.wait()
        for r in range(16):
            w0 = w0_v[c * 16 + r, :]
            w1 = w1_v[c * 16 + r, :]

            def body(j, _, r=r, w0=w0, w1=w1):
                sl = pl.ds(j * 16, 16)
                o_v[r, sl] = w0 * r0_v[r, sl] + w1 * r1_v[r, sl]
                return 0

            lax.fori_loop(0, H // 16, body, 0)
        pltpu.sync_copy(o_v, out_hbm.at[pl.ds(base_t + c * 16, 16)])


def _combine(yd, p0, p1, w0r, w1r):
    mesh = plsc.VectorSubcoreMesh(core_axis_name="c", subcore_axis_name="s")
    f = functools.partial(
        pl.kernel,
        out_type=jax.ShapeDtypeStruct((T, H), jnp.float32),
        mesh=mesh,
        scratch_types=[
            pltpu.VMEM((NCH, 16), jnp.int32),
            pltpu.VMEM((NCH, 16), jnp.int32),
            pltpu.VMEM((TPW, 16), jnp.float32),
            pltpu.VMEM((TPW, 16), jnp.float32),
            pltpu.VMEM((16, H), jnp.float32),
            pltpu.VMEM((16, H), jnp.float32),
            pltpu.VMEM((16, H), jnp.float32),
            pltpu.SemaphoreType.DMA,
            pltpu.SemaphoreType.DMA,
        ],
    )(_comb_body)
    return f(yd, p0, p1, w0r, w1r)


def kernel(hidden_states, gate_w, w13_weight, w2_weight):
    x = hidden_states
    pos, w0b, w1b, be = _routing(x, gate_w)
    be_flat = be.reshape(NB + 8)
    p0 = pos[:, 0].reshape(NW, NCH, 16)
    p1 = pos[:, 1].reshape(NW, NCH, 16)
    w0r = w0b.reshape(NW, TPW, 16)
    w1r = w1b.reshape(NW, TPW, 16)
    xd = _dispatch(x, p0, p1)
    yd = _grouped_mm(be_flat, xd, w13_weight, w2_weight)
    out = _combine(yd, p0, p1, w0r, w1r)
    return out


# speedup vs baseline: 2.0601x; 1.0261x over previous
import functools

import jax
import jax.numpy as jnp
from jax import lax
from jax.experimental import pallas as pl
from jax.experimental.pallas import tpu as pltpu
from jax.experimental.pallas import tpu_sc as plsc

E = 8
K = 2
T = 2048
H = 1024
I = 1792
BLK = 256
NB = 24
NPAD = NB * BLK
IT = 896
NI = I // IT

NW = 32
TPW = T // NW
NCH = TPW // 16


def _routing_body(x_ref, gw_ref, pos_ref, w0b_ref, w1b_ref, be_ref):
    x = x_ref[...]
    gw = gw_ref[...]
    logits = lax.dot_general(x, gw, (((1,), (1,)), ((), ())),
                             preferred_element_type=jnp.float32)
    m = jnp.max(logits, axis=-1, keepdims=True)
    ex = jnp.exp(logits - m)
    p = ex / jnp.sum(ex, axis=-1, keepdims=True)

    eidx = lax.broadcasted_iota(jnp.int32, (T, E), 1)
    m1 = jnp.max(p, axis=-1, keepdims=True)
    idx1 = jnp.min(jnp.where(p == m1, eidx, E), axis=-1, keepdims=True)
    oh0 = eidx == idx1
    pm = jnp.where(oh0, -jnp.inf, p)
    m2 = jnp.max(pm, axis=-1, keepdims=True)
    idx2 = jnp.min(jnp.where(pm == m2, eidx, E), axis=-1, keepdims=True)
    oh1 = eidx == idx2

    s = m1 + m2
    w0 = m1 / s
    w1 = m2 / s

    oh01 = oh0.astype(jnp.int32) + oh1.astype(jnp.int32)
    acc = oh01
    sh = 1
    while sh < T:
        shifted = jnp.concatenate(
            [jnp.zeros((sh, E), jnp.int32), acc[:T - sh]], axis=0)
        acc = acc + shifted
        sh *= 2
    C = acc - oh01

    cnt = jnp.sum(oh01, axis=0, keepdims=True).astype(jnp.float32)
    padded = jnp.ceil(cnt / BLK) * BLK
    tril = (lax.broadcasted_iota(jnp.int32, (E, E), 0)
            <= lax.broadcasted_iota(jnp.int32, (E, E), 1)).astype(jnp.float32)
    cum = lax.dot_general(padded, tril, (((1,), (0,)), ((), ())),
                          preferred_element_type=jnp.float32)
    pad_off = cum - padded

    rank0 = jnp.sum(jnp.where(oh0, C, 0), axis=-1, keepdims=True)
    rank1 = jnp.sum(jnp.where(oh1, C, 0), axis=-1, keepdims=True)
    off0 = jnp.sum(jnp.where(oh0, pad_off, 0.0), axis=-1, keepdims=True)
    off1 = jnp.sum(jnp.where(oh1, pad_off, 0.0), axis=-1, keepdims=True)
    pos0 = rank0 + off0.astype(jnp.int32)
    pos1 = rank1 + off1.astype(jnp.int32)
    pos_ref[...] = jnp.concatenate([pos0, pos1], axis=1)

    w0b_ref[...] = jnp.broadcast_to(w0, (T, 16))
    w1b_ref[...] = jnp.broadcast_to(w1, (T, 16))

    brow = (lax.broadcasted_iota(jnp.int32, (NB + 8, E), 0) * BLK).astype(
        jnp.float32)
    cmp = brow >= jnp.broadcast_to(cum, (NB + 8, E))
    be = jnp.sum(cmp.astype(jnp.int32), axis=-1, keepdims=True)
    eiota = lax.broadcasted_iota(jnp.int32, (1, E), 1)
    maxe = jnp.max(jnp.where(cnt > 0, eiota, 0), axis=-1, keepdims=True)
    nact = (cum[:, E - 1:E] / BLK).astype(jnp.int32)
    biota = lax.broadcasted_iota(jnp.int32, (NB + 8, 1), 0)
    be = jnp.minimum(be, jnp.broadcast_to(maxe, (NB + 8, 1)))
    be_ref[...] = jnp.where(biota == NB, jnp.broadcast_to(nact, (NB + 8, 1)),
                            be)


def _routing(x, gate_w):
    return pl.pallas_call(
        _routing_body,
        out_shape=[
            jax.ShapeDtypeStruct((T, K), jnp.int32),
            jax.ShapeDtypeStruct((T, 16), jnp.float32),
            jax.ShapeDtypeStruct((T, 16), jnp.float32),
            jax.ShapeDtypeStruct((NB + 8, 1), jnp.int32),
        ],
    )(x, gate_w)


def _disp_body(x_hbm, p0_hbm, p1_hbm, xd_hbm, p0_v, p1_v, rows_a, rows_b,
               sem_r, sem_s):
    wid = lax.axis_index("s") * 2 + lax.axis_index("c")
    base_t = wid * TPW
    pltpu.sync_copy(p0_hbm.at[wid], p0_v)
    pltpu.sync_copy(p1_hbm.at[wid], p1_v)
    bufs = [rows_a, rows_b]
    reads = [None] * NCH
    reads[0] = pltpu.async_copy(x_hbm.at[pl.ds(base_t, 16)], rows_a, sem_r)
    scat = []
    for c in range(NCH):
        cur = bufs[c % 2]
        reads[c].wait()
        if c + 1 < NCH:
            for s in scat:
                s.wait()
            scat = []
            reads[c + 1] = pltpu.async_copy(
                x_hbm.at[pl.ds(base_t + (c + 1) * 16, 16)], bufs[(c + 1) % 2],
                sem_r)
        scat.append(pltpu.async_copy(cur, xd_hbm.at[p0_v.at[c]], sem_s))
        scat.append(pltpu.async_copy(cur, xd_hbm.at[p1_v.at[c]], sem_s))
    for s in scat:
        s.wait()


def _dispatch(x, p0, p1):
    mesh = plsc.VectorSubcoreMesh(core_axis_name="c", subcore_axis_name="s")
    f = functools.partial(
        pl.kernel,
        out_type=jax.ShapeDtypeStruct((NPAD, H), jnp.float32),
        mesh=mesh,
        scratch_types=[
            pltpu.VMEM((NCH, 16), jnp.int32),
            pltpu.VMEM((NCH, 16), jnp.int32),
            pltpu.VMEM((16, H), jnp.float32),
            pltpu.VMEM((16, H), jnp.float32),
            pltpu.SemaphoreType.DMA,
            pltpu.SemaphoreType.DMA,
        ],
    )(_disp_body)
    return f(x, p0, p1)


def _mm_body(be_ref, xd_ref, w13g_ref, w13u_ref, w2_ref, out_ref):
    b = pl.program_id(0)
    nact = be_ref[NB]

    @pl.when(b < nact)
    def _():
        xb = xd_ref[...].astype(jnp.bfloat16)
        g = lax.dot_general(
            xb, w13g_ref[0].astype(jnp.bfloat16), (((1,), (1,)), ((), ())),
            preferred_element_type=jnp.float32)
        u = lax.dot_general(
            xb, w13u_ref[0].astype(jnp.bfloat16), (((1,), (1,)), ((), ())),
            preferred_element_type=jnp.float32)
        act = (g * jax.nn.sigmoid(g) * u).astype(jnp.bfloat16)
        out_ref[...] = lax.dot_general(
            act, w2_ref[0].astype(jnp.bfloat16), (((1,), (1,)), ((), ())),
            preferred_element_type=jnp.float32)


def _grouped_mm(be, xd, w13, w2):
    grid_spec = pltpu.PrefetchScalarGridSpec(
        num_scalar_prefetch=1,
        grid=(NB,),
        in_specs=[
            pl.BlockSpec((BLK, H),
                         lambda b, be_r: (jnp.where(b < be_r[NB], b, 0), 0)),
            pl.BlockSpec((1, I, H), lambda b, be_r: (be_r[b], 0, 0)),
            pl.BlockSpec((1, I, H), lambda b, be_r: (be_r[b], 1, 0)),
            pl.BlockSpec((1, H, I), lambda b, be_r: (be_r[b], 0, 0)),
        ],
        out_specs=pl.BlockSpec(
            (BLK, H),
            lambda b, be_r: (jnp.where(b < be_r[NB], b, be_r[NB]), 0)),
    )
    return pl.pallas_call(
        _mm_body,
        grid_spec=grid_spec,
        out_shape=jax.ShapeDtypeStruct((NPAD, H), jnp.float32),
        compiler_params=pltpu.CompilerParams(
            dimension_semantics=("arbitrary",)),
    )(be, xd, w13, w13, w2)


def _comb_body(yd_hbm, p0_hbm, p1_hbm, w0_hbm, w1_hbm, out_hbm,
               p0_v, p1_v, w0_v, w1_v, r0_v, r1_v, o_v, sem0, sem1):
    wid = lax.axis_index("s") * 2 + lax.axis_index("c")
    base_t = wid * TPW
    pltpu.sync_copy(p0_hbm.at[wid], p0_v)
    pltpu.sync_copy(p1_hbm.at[wid], p1_v)
    pltpu.sync_copy(w0_hbm.at[wid], w0_v)
    pltpu.sync_copy(w1_hbm.at[wid], w1_v)
    for c in range(NCH):
        g0 = pltpu.async_copy(yd_hbm.at[p0_v.at[c]], r0_v, sem0)
        g1 = pltpu.async_copy(yd_hbm.at[p1_v.at[c]], r1_v, sem1)
        g0.wait()
        g1.wait()
        for r in range(16):
            w0 = w0_v[c * 16 + r, :]
            w1 = w1_v[c * 16 + r, :]

            def body(j, _, r=r, w0=w0, w1=w1):
                sl = pl.ds(j * 16, 16)
                o_v[r, sl] = w0 * r0_v[r, sl] + w1 * r1_v[r, sl]
                return 0

            lax.fori_loop(0, H // 16, body, 0)
        pltpu.sync_copy(o_v, out_hbm.at[pl.ds(base_t + c * 16, 16)])


def _combine(yd, p0, p1, w0r, w1r):
    mesh = plsc.VectorSubcoreMesh(core_axis_name="c", subcore_axis_name="s")
    f = functools.partial(
        pl.kernel,
        out_type=jax.ShapeDtypeStruct((T, H), jnp.float32),
        mesh=mesh,
        scratch_types=[
            pltpu.VMEM((NCH, 16), jnp.int32),
            pltpu.VMEM((NCH, 16), jnp.int32),
            pltpu.VMEM((TPW, 16), jnp.float32),
            pltpu.VMEM((TPW, 16), jnp.float32),
            pltpu.VMEM((16, H), jnp.float32),
            pltpu.VMEM((16, H), jnp.float32),
            pltpu.VMEM((16, H), jnp.float32),
            pltpu.SemaphoreType.DMA,
            pltpu.SemaphoreType.DMA,
        ],
    )(_comb_body)
    return f(yd, p0, p1, w0r, w1r)


def kernel(hidden_states, gate_w, w13_weight, w2_weight):
    x = hidden_states
    pos, w0b, w1b, be = _routing(x, gate_w)
    be_flat = be.reshape(NB + 8)
    p0 = pos[:, 0].reshape(NW, NCH, 16)
    p1 = pos[:, 1].reshape(NW, NCH, 16)
    w0r = w0b.reshape(NW, TPW, 16)
    w1r = w1b.reshape(NW, TPW, 16)
    xd = _dispatch(x, p0, p1)
    yd = _grouped_mm(be_flat, xd, w13_weight, w2_weight)
    out = _combine(yd, p0, p1, w0r, w1r)
    return out


# speedup vs baseline: 2.1561x; 1.0466x over previous
import functools

import jax
import jax.numpy as jnp
from jax import lax
from jax.experimental import pallas as pl
from jax.experimental.pallas import tpu as pltpu
from jax.experimental.pallas import tpu_sc as plsc

E = 8
K = 2
T = 2048
H = 1024
I = 1792
BLK = 256
NB = 24
NPAD = NB * BLK
IT = 896
NI = I // IT

NW = 32
TPW = T // NW
NCH = TPW // 16


def _routing_body(x_ref, gw_ref, pos_ref, w0b_ref, w1b_ref, be_ref):
    x = x_ref[...]
    gw = gw_ref[...]
    logits = lax.dot_general(x, gw, (((1,), (1,)), ((), ())),
                             preferred_element_type=jnp.float32)
    m = jnp.max(logits, axis=-1, keepdims=True)
    ex = jnp.exp(logits - m)
    p = ex / jnp.sum(ex, axis=-1, keepdims=True)

    eidx = lax.broadcasted_iota(jnp.int32, (T, E), 1)
    m1 = jnp.max(p, axis=-1, keepdims=True)
    idx1 = jnp.min(jnp.where(p == m1, eidx, E), axis=-1, keepdims=True)
    oh0 = eidx == idx1
    pm = jnp.where(oh0, -jnp.inf, p)
    m2 = jnp.max(pm, axis=-1, keepdims=True)
    idx2 = jnp.min(jnp.where(pm == m2, eidx, E), axis=-1, keepdims=True)
    oh1 = eidx == idx2

    s = m1 + m2
    w0 = m1 / s
    w1 = m2 / s

    oh01 = oh0.astype(jnp.int32) + oh1.astype(jnp.int32)
    acc = oh01
    sh = 1
    while sh < T:
        shifted = jnp.concatenate(
            [jnp.zeros((sh, E), jnp.int32), acc[:T - sh]], axis=0)
        acc = acc + shifted
        sh *= 2
    C = acc - oh01

    cnt = jnp.sum(oh01, axis=0, keepdims=True).astype(jnp.float32)
    padded = jnp.ceil(cnt / BLK) * BLK
    tril = (lax.broadcasted_iota(jnp.int32, (E, E), 0)
            <= lax.broadcasted_iota(jnp.int32, (E, E), 1)).astype(jnp.float32)
    cum = lax.dot_general(padded, tril, (((1,), (0,)), ((), ())),
                          preferred_element_type=jnp.float32)
    pad_off = cum - padded

    rank0 = jnp.sum(jnp.where(oh0, C, 0), axis=-1, keepdims=True)
    rank1 = jnp.sum(jnp.where(oh1, C, 0), axis=-1, keepdims=True)
    off0 = jnp.sum(jnp.where(oh0, pad_off, 0.0), axis=-1, keepdims=True)
    off1 = jnp.sum(jnp.where(oh1, pad_off, 0.0), axis=-1, keepdims=True)
    pos0 = rank0 + off0.astype(jnp.int32)
    pos1 = rank1 + off1.astype(jnp.int32)
    pos_ref[...] = jnp.concatenate([pos0, pos1], axis=1)

    w0b_ref[...] = jnp.broadcast_to(w0, (T, 16))
    w1b_ref[...] = jnp.broadcast_to(w1, (T, 16))

    brow = (lax.broadcasted_iota(jnp.int32, (NB + 8, E), 0) * BLK).astype(
        jnp.float32)
    cmp = brow >= jnp.broadcast_to(cum, (NB + 8, E))
    be = jnp.sum(cmp.astype(jnp.int32), axis=-1, keepdims=True)
    eiota = lax.broadcasted_iota(jnp.int32, (1, E), 1)
    maxe = jnp.max(jnp.where(cnt > 0, eiota, 0), axis=-1, keepdims=True)
    nact = (cum[:, E - 1:E] / BLK).astype(jnp.int32)
    biota = lax.broadcasted_iota(jnp.int32, (NB + 8, 1), 0)
    be = jnp.minimum(be, jnp.broadcast_to(maxe, (NB + 8, 1)))
    be_ref[...] = jnp.where(biota == NB, jnp.broadcast_to(nact, (NB + 8, 1)),
                            be)


def _routing(x, gate_w):
    return pl.pallas_call(
        _routing_body,
        out_shape=[
            jax.ShapeDtypeStruct((T, K), jnp.int32),
            jax.ShapeDtypeStruct((T, 16), jnp.float32),
            jax.ShapeDtypeStruct((T, 16), jnp.float32),
            jax.ShapeDtypeStruct((NB + 8, 1), jnp.int32),
        ],
    )(x, gate_w)


def _disp_body(x_hbm, p0_hbm, p1_hbm, xd_hbm, p0_v, p1_v, rows_a, rows_b,
               sem_r, sem_s):
    wid = lax.axis_index("s") * 2 + lax.axis_index("c")
    base_t = wid * TPW
    pltpu.sync_copy(p0_hbm.at[wid], p0_v)
    pltpu.sync_copy(p1_hbm.at[wid], p1_v)
    bufs = [rows_a, rows_b]
    reads = [None] * NCH
    reads[0] = pltpu.async_copy(x_hbm.at[pl.ds(base_t, 16)], rows_a, sem_r)
    scat = []
    for c in range(NCH):
        cur = bufs[c % 2]
        reads[c].wait()
        if c + 1 < NCH:
            for s in scat:
                s.wait()
            scat = []
            reads[c + 1] = pltpu.async_copy(
                x_hbm.at[pl.ds(base_t + (c + 1) * 16, 16)], bufs[(c + 1) % 2],
                sem_r)
        scat.append(pltpu.async_copy(cur, xd_hbm.at[p0_v.at[c]], sem_s))
        scat.append(pltpu.async_copy(cur, xd_hbm.at[p1_v.at[c]], sem_s))
    for s in scat:
        s.wait()


def _dispatch(x, p0, p1):
    mesh = plsc.VectorSubcoreMesh(core_axis_name="c", subcore_axis_name="s")
    f = functools.partial(
        pl.kernel,
        out_type=jax.ShapeDtypeStruct((NPAD, H), jnp.float32),
        mesh=mesh,
        scratch_types=[
            pltpu.VMEM((NCH, 16), jnp.int32),
            pltpu.VMEM((NCH, 16), jnp.int32),
            pltpu.VMEM((16, H), jnp.float32),
            pltpu.VMEM((16, H), jnp.float32),
            pltpu.SemaphoreType.DMA,
            pltpu.SemaphoreType.DMA,
        ],
    )(_disp_body)
    return f(x, p0, p1)


def _mm_body(be_ref, xd_ref, w13g_ref, w13u_ref, w2_ref, out_ref):
    b = pl.program_id(0)
    nact = be_ref[NB]

    @pl.when(b < nact)
    def _():
        xb = xd_ref[...].astype(jnp.bfloat16)
        g = lax.dot_general(
            xb, w13g_ref[0].astype(jnp.bfloat16), (((1,), (1,)), ((), ())),
            preferred_element_type=jnp.float32)
        u = lax.dot_general(
            xb, w13u_ref[0].astype(jnp.bfloat16), (((1,), (1,)), ((), ())),
            preferred_element_type=jnp.float32)
        act = (g * jax.nn.sigmoid(g) * u).astype(jnp.bfloat16)
        out_ref[...] = lax.dot_general(
            act, w2_ref[0].astype(jnp.bfloat16), (((1,), (1,)), ((), ())),
            preferred_element_type=jnp.float32)


def _grouped_mm(be, xd, w13, w2):
    grid_spec = pltpu.PrefetchScalarGridSpec(
        num_scalar_prefetch=1,
        grid=(NB,),
        in_specs=[
            pl.BlockSpec((BLK, H),
                         lambda b, be_r: (jnp.where(b < be_r[NB], b, 0), 0)),
            pl.BlockSpec((1, I, H), lambda b, be_r: (be_r[b], 0, 0)),
            pl.BlockSpec((1, I, H), lambda b, be_r: (be_r[b], 1, 0)),
            pl.BlockSpec((1, H, I), lambda b, be_r: (be_r[b], 0, 0)),
        ],
        out_specs=pl.BlockSpec(
            (BLK, H),
            lambda b, be_r: (jnp.where(b < be_r[NB], b, be_r[NB]), 0)),
    )
    return pl.pallas_call(
        _mm_body,
        grid_spec=grid_spec,
        out_shape=jax.ShapeDtypeStruct((NPAD, H), jnp.float32),
        compiler_params=pltpu.CompilerParams(
            dimension_semantics=("arbitrary",)),
    )(be, xd, w13, w13, w2)


def _comb_body(yd_hbm, p0_hbm, p1_hbm, w0_hbm, w1_hbm, out_hbm,
               p0_v, p1_v, w0_v, w1_v, r0a, r0b, r1a, r1b, oa, ob,
               sem_g, sem_w):
    wid = lax.axis_index("s") * 2 + lax.axis_index("c")
    base_t = wid * TPW
    pltpu.sync_copy(p0_hbm.at[wid], p0_v)
    pltpu.sync_copy(p1_hbm.at[wid], p1_v)
    pltpu.sync_copy(w0_hbm.at[wid], w0_v)
    pltpu.sync_copy(w1_hbm.at[wid], w1_v)
    r0s = [r0a, r0b]
    r1s = [r1a, r1b]
    os_ = [oa, ob]
    g0 = [None] * NCH
    g1 = [None] * NCH
    wr = [None] * NCH
    g0[0] = pltpu.async_copy(yd_hbm.at[p0_v.at[0]], r0a, sem_g)
    g1[0] = pltpu.async_copy(yd_hbm.at[p1_v.at[0]], r1a, sem_g)
    for c in range(NCH):
        if c + 1 < NCH:
            g0[c + 1] = pltpu.async_copy(
                yd_hbm.at[p0_v.at[c + 1]], r0s[(c + 1) % 2], sem_g)
            g1[c + 1] = pltpu.async_copy(
                yd_hbm.at[p1_v.at[c + 1]], r1s[(c + 1) % 2], sem_g)
        g0[c].wait()
        g1[c].wait()
        if c >= 2:
            wr[c - 2].wait()
        r0_v = r0s[c % 2]
        r1_v = r1s[c % 2]
        o_v = os_[c % 2]
        for r in range(16):
            w0 = w0_v[c * 16 + r, :]
            w1 = w1_v[c * 16 + r, :]

            def body(j, _, r=r, w0=w0, w1=w1, r0_v=r0_v, r1_v=r1_v, o_v=o_v):
                sl = pl.ds(j * 16, 16)
                o_v[r, sl] = w0 * r0_v[r, sl] + w1 * r1_v[r, sl]
                return 0

            lax.fori_loop(0, H // 16, body, 0)
        wr[c] = pltpu.async_copy(
            o_v, out_hbm.at[pl.ds(base_t + c * 16, 16)], sem_w)
    wr[NCH - 2].wait()
    wr[NCH - 1].wait()


def _combine(yd, p0, p1, w0r, w1r):
    mesh = plsc.VectorSubcoreMesh(core_axis_name="c", subcore_axis_name="s")
    f = functools.partial(
        pl.kernel,
        out_type=jax.ShapeDtypeStruct((T, H), jnp.float32),
        mesh=mesh,
        scratch_types=[
            pltpu.VMEM((NCH, 16), jnp.int32),
            pltpu.VMEM((NCH, 16), jnp.int32),
            pltpu.VMEM((TPW, 16), jnp.float32),
            pltpu.VMEM((TPW, 16), jnp.float32),
            pltpu.VMEM((16, H), jnp.float32),
            pltpu.VMEM((16, H), jnp.float32),
            pltpu.VMEM((16, H), jnp.float32),
            pltpu.VMEM((16, H), jnp.float32),
            pltpu.VMEM((16, H), jnp.float32),
            pltpu.VMEM((16, H), jnp.float32),
            pltpu.SemaphoreType.DMA,
            pltpu.SemaphoreType.DMA,
        ],
    )(_comb_body)
    return f(yd, p0, p1, w0r, w1r)


def kernel(hidden_states, gate_w, w13_weight, w2_weight):
    x = hidden_states
    pos, w0b, w1b, be = _routing(x, gate_w)
    be_flat = be.reshape(NB + 8)
    p0 = pos[:, 0].reshape(NW, NCH, 16)
    p1 = pos[:, 1].reshape(NW, NCH, 16)
    w0r = w0b.reshape(NW, TPW, 16)
    w1r = w1b.reshape(NW, TPW, 16)
    xd = _dispatch(x, p0, p1)
    yd = _grouped_mm(be_flat, xd, w13_weight, w2_weight)
    out = _combine(yd, p0, p1, w0r, w1r)
    return out


# speedup vs baseline: 2.1947x; 1.0179x over previous
import functools

import jax
import jax.numpy as jnp
from jax import lax
from jax.experimental import pallas as pl
from jax.experimental.pallas import tpu as pltpu
from jax.experimental.pallas import tpu_sc as plsc

E = 8
K = 2
T = 2048
H = 1024
I = 1792
BLK = 256
NB = 24
NPAD = NB * BLK
IT = 896
NI = I // IT

NW = 32
TPW = T // NW
NCH = TPW // 16


def _routing_body(x_ref, gw_ref, pos_ref, w0b_ref, w1b_ref, be_ref, xp_ref):
    x = x_ref[...]
    xb = x.astype(jnp.bfloat16)
    hi = lax.bitcast_convert_type(xb[:, :H // 2], jnp.uint16).astype(
        jnp.int32)
    lo = lax.bitcast_convert_type(xb[:, H // 2:], jnp.uint16).astype(
        jnp.int32)
    xp_ref[...] = (hi << 16) | lo
    gw = gw_ref[...]
    logits = lax.dot_general(x, gw, (((1,), (1,)), ((), ())),
                             preferred_element_type=jnp.float32)
    m = jnp.max(logits, axis=-1, keepdims=True)
    ex = jnp.exp(logits - m)
    p = ex / jnp.sum(ex, axis=-1, keepdims=True)

    eidx = lax.broadcasted_iota(jnp.int32, (T, E), 1)
    m1 = jnp.max(p, axis=-1, keepdims=True)
    idx1 = jnp.min(jnp.where(p == m1, eidx, E), axis=-1, keepdims=True)
    oh0 = eidx == idx1
    pm = jnp.where(oh0, -jnp.inf, p)
    m2 = jnp.max(pm, axis=-1, keepdims=True)
    idx2 = jnp.min(jnp.where(pm == m2, eidx, E), axis=-1, keepdims=True)
    oh1 = eidx == idx2

    s = m1 + m2
    w0 = m1 / s
    w1 = m2 / s

    oh01 = oh0.astype(jnp.int32) + oh1.astype(jnp.int32)
    acc = oh01
    sh = 1
    while sh < T:
        shifted = jnp.concatenate(
            [jnp.zeros((sh, E), jnp.int32), acc[:T - sh]], axis=0)
        acc = acc + shifted
        sh *= 2
    C = acc - oh01

    cnt = jnp.sum(oh01, axis=0, keepdims=True).astype(jnp.float32)
    padded = jnp.ceil(cnt / BLK) * BLK
    tril = (lax.broadcasted_iota(jnp.int32, (E, E), 0)
            <= lax.broadcasted_iota(jnp.int32, (E, E), 1)).astype(jnp.float32)
    cum = lax.dot_general(padded, tril, (((1,), (0,)), ((), ())),
                          preferred_element_type=jnp.float32)
    pad_off = cum - padded

    rank0 = jnp.sum(jnp.where(oh0, C, 0), axis=-1, keepdims=True)
    rank1 = jnp.sum(jnp.where(oh1, C, 0), axis=-1, keepdims=True)
    off0 = jnp.sum(jnp.where(oh0, pad_off, 0.0), axis=-1, keepdims=True)
    off1 = jnp.sum(jnp.where(oh1, pad_off, 0.0), axis=-1, keepdims=True)
    pos0 = rank0 + off0.astype(jnp.int32)
    pos1 = rank1 + off1.astype(jnp.int32)
    pos_ref[...] = jnp.concatenate([pos0, pos1], axis=1)

    w0b_ref[...] = jnp.broadcast_to(w0, (T, 16))
    w1b_ref[...] = jnp.broadcast_to(w1, (T, 16))

    brow = (lax.broadcasted_iota(jnp.int32, (NB + 8, E), 0) * BLK).astype(
        jnp.float32)
    cmp = brow >= jnp.broadcast_to(cum, (NB + 8, E))
    be = jnp.sum(cmp.astype(jnp.int32), axis=-1, keepdims=True)
    eiota = lax.broadcasted_iota(jnp.int32, (1, E), 1)
    maxe = jnp.max(jnp.where(cnt > 0, eiota, 0), axis=-1, keepdims=True)
    nact = (cum[:, E - 1:E] / BLK).astype(jnp.int32)
    biota = lax.broadcasted_iota(jnp.int32, (NB + 8, 1), 0)
    be = jnp.minimum(be, jnp.broadcast_to(maxe, (NB + 8, 1)))
    be_ref[...] = jnp.where(biota == NB, jnp.broadcast_to(nact, (NB + 8, 1)),
                            be)


def _routing(x, gate_w):
    return pl.pallas_call(
        _routing_body,
        out_shape=[
            jax.ShapeDtypeStruct((T, K), jnp.int32),
            jax.ShapeDtypeStruct((T, 16), jnp.float32),
            jax.ShapeDtypeStruct((T, 16), jnp.float32),
            jax.ShapeDtypeStruct((NB + 8, 1), jnp.int32),
            jax.ShapeDtypeStruct((T, H // 2), jnp.int32),
        ],
    )(x, gate_w)


def _disp_body(x_hbm, p0_hbm, p1_hbm, xd_hbm, p0_v, p1_v, rows_a, rows_b,
               sem_r, sem_s):
    wid = lax.axis_index("s") * 2 + lax.axis_index("c")
    base_t = wid * TPW
    pltpu.sync_copy(p0_hbm.at[wid], p0_v)
    pltpu.sync_copy(p1_hbm.at[wid], p1_v)
    bufs = [rows_a, rows_b]
    reads = [None] * NCH
    reads[0] = pltpu.async_copy(x_hbm.at[pl.ds(base_t, 16)], rows_a, sem_r)
    scat = []
    for c in range(NCH):
        cur = bufs[c % 2]
        reads[c].wait()
        if c + 1 < NCH:
            for s in scat:
                s.wait()
            scat = []
            reads[c + 1] = pltpu.async_copy(
                x_hbm.at[pl.ds(base_t + (c + 1) * 16, 16)], bufs[(c + 1) % 2],
                sem_r)
        scat.append(pltpu.async_copy(cur, xd_hbm.at[p0_v.at[c]], sem_s))
        scat.append(pltpu.async_copy(cur, xd_hbm.at[p1_v.at[c]], sem_s))
    for s in scat:
        s.wait()


def _dispatch(x, p0, p1):
    mesh = plsc.VectorSubcoreMesh(core_axis_name="c", subcore_axis_name="s")
    f = functools.partial(
        pl.kernel,
        out_type=jax.ShapeDtypeStruct((NPAD, H // 2), jnp.int32),
        mesh=mesh,
        scratch_types=[
            pltpu.VMEM((NCH, 16), jnp.int32),
            pltpu.VMEM((NCH, 16), jnp.int32),
            pltpu.VMEM((16, H // 2), jnp.int32),
            pltpu.VMEM((16, H // 2), jnp.int32),
            pltpu.SemaphoreType.DMA,
            pltpu.SemaphoreType.DMA,
        ],
    )(_disp_body)
    return f(x, p0, p1)


def _mm_body(be_ref, xd_ref, w13g_ref, w13u_ref, w2_ref, out_ref):
    b = pl.program_id(0)
    nact = be_ref[NB]

    @pl.when(b < nact)
    def _():
        w = xd_ref[...]
        xh = lax.bitcast_convert_type(
            lax.shift_right_logical(w, 16).astype(jnp.uint16), jnp.bfloat16)
        xl = lax.bitcast_convert_type(
            jnp.bitwise_and(w, 0xFFFF).astype(jnp.uint16), jnp.bfloat16)
        xb = jnp.concatenate([xh, xl], axis=1)
        g = lax.dot_general(
            xb, w13g_ref[0].astype(jnp.bfloat16), (((1,), (1,)), ((), ())),
            preferred_element_type=jnp.float32)
        u = lax.dot_general(
            xb, w13u_ref[0].astype(jnp.bfloat16), (((1,), (1,)), ((), ())),
            preferred_element_type=jnp.float32)
        act = (g * jax.nn.sigmoid(g) * u).astype(jnp.bfloat16)
        out_ref[...] = lax.dot_general(
            act, w2_ref[0].astype(jnp.bfloat16), (((1,), (1,)), ((), ())),
            preferred_element_type=jnp.float32)


def _grouped_mm(be, xd, w13, w2):
    grid_spec = pltpu.PrefetchScalarGridSpec(
        num_scalar_prefetch=1,
        grid=(NB,),
        in_specs=[
            pl.BlockSpec((BLK, H // 2),
                         lambda b, be_r: (jnp.where(b < be_r[NB], b, 0), 0)),
            pl.BlockSpec((1, I, H), lambda b, be_r: (be_r[b], 0, 0)),
            pl.BlockSpec((1, I, H), lambda b, be_r: (be_r[b], 1, 0)),
            pl.BlockSpec((1, H, I), lambda b, be_r: (be_r[b], 0, 0)),
        ],
        out_specs=pl.BlockSpec(
            (BLK, H),
            lambda b, be_r: (jnp.where(b < be_r[NB], b, be_r[NB]), 0)),
    )
    return pl.pallas_call(
        _mm_body,
        grid_spec=grid_spec,
        out_shape=jax.ShapeDtypeStruct((NPAD, H), jnp.float32),
        compiler_params=pltpu.CompilerParams(
            dimension_semantics=("arbitrary",)),
    )(be, xd, w13, w13, w2)


def _comb_body(yd_hbm, p0_hbm, p1_hbm, w0_hbm, w1_hbm, out_hbm,
               p0_v, p1_v, w0_v, w1_v, r0a, r0b, r1a, r1b, oa, ob,
               sem_g, sem_w):
    wid = lax.axis_index("s") * 2 + lax.axis_index("c")
    base_t = wid * TPW
    pltpu.sync_copy(p0_hbm.at[wid], p0_v)
    pltpu.sync_copy(p1_hbm.at[wid], p1_v)
    pltpu.sync_copy(w0_hbm.at[wid], w0_v)
    pltpu.sync_copy(w1_hbm.at[wid], w1_v)
    r0s = [r0a, r0b]
    r1s = [r1a, r1b]
    os_ = [oa, ob]
    g0 = [None] * NCH
    g1 = [None] * NCH
    wr = [None] * NCH
    g0[0] = pltpu.async_copy(yd_hbm.at[p0_v.at[0]], r0a, sem_g)
    g1[0] = pltpu.async_copy(yd_hbm.at[p1_v.at[0]], r1a, sem_g)
    for c in range(NCH):
        if c + 1 < NCH:
            g0[c + 1] = pltpu.async_copy(
                yd_hbm.at[p0_v.at[c + 1]], r0s[(c + 1) % 2], sem_g)
            g1[c + 1] = pltpu.async_copy(
                yd_hbm.at[p1_v.at[c + 1]], r1s[(c + 1) % 2], sem_g)
        g0[c].wait()
        g1[c].wait()
        if c >= 2:
            wr[c - 2].wait()
        r0_v = r0s[c % 2]
        r1_v = r1s[c % 2]
        o_v = os_[c % 2]
        for r in range(16):
            w0 = w0_v[c * 16 + r, :]
            w1 = w1_v[c * 16 + r, :]

            def body(j, _, r=r, w0=w0, w1=w1, r0_v=r0_v, r1_v=r1_v, o_v=o_v):
                sl = pl.ds(j * 16, 16)
                o_v[r, sl] = w0 * r0_v[r, sl] + w1 * r1_v[r, sl]
                return 0

            lax.fori_loop(0, H // 16, body, 0)
        wr[c] = pltpu.async_copy(
            o_v, out_hbm.at[pl.ds(base_t + c * 16, 16)], sem_w)
    wr[NCH - 2].wait()
    wr[NCH - 1].wait()


def _combine(yd, p0, p1, w0r, w1r):
    mesh = plsc.VectorSubcoreMesh(core_axis_name="c", subcore_axis_name="s")
    f = functools.partial(
        pl.kernel,
        out_type=jax.ShapeDtypeStruct((T, H), jnp.float32),
        mesh=mesh,
        scratch_types=[
            pltpu.VMEM((NCH, 16), jnp.int32),
            pltpu.VMEM((NCH, 16), jnp.int32),
            pltpu.VMEM((TPW, 16), jnp.float32),
            pltpu.VMEM((TPW, 16), jnp.float32),
            pltpu.VMEM((16, H), jnp.float32),
            pltpu.VMEM((16, H), jnp.float32),
            pltpu.VMEM((16, H), jnp.float32),
            pltpu.VMEM((16, H), jnp.float32),
            pltpu.VMEM((16, H), jnp.float32),
            pltpu.VMEM((16, H), jnp.float32),
            pltpu.SemaphoreType.DMA,
            pltpu.SemaphoreType.DMA,
        ],
    )(_comb_body)
    return f(yd, p0, p1, w0r, w1r)


def kernel(hidden_states, gate_w, w13_weight, w2_weight):
    x = hidden_states
    pos, w0b, w1b, be, xp = _routing(x, gate_w)
    be_flat = be.reshape(NB + 8)
    p0 = pos[:, 0].reshape(NW, NCH, 16)
    p1 = pos[:, 1].reshape(NW, NCH, 16)
    w0r = w0b.reshape(NW, TPW, 16)
    w1r = w1b.reshape(NW, TPW, 16)
    xd = _dispatch(xp, p0, p1)
    yd = _grouped_mm(be_flat, xd, w13_weight, w2_weight)
    out = _combine(yd, p0, p1, w0r, w1r)
    return out
